# Initial kernel scaffold; baseline (speedup 1.0000x reference)
import jax
import jax.numpy as jnp
from jax.experimental import pallas as pl


def kernel(X, edge_index, edge_weight, skip, H, C, Wx0, Wx1, Wh0, Wh1, b, w_peep, ln_g, ln_b, fc_w, fc_b):
    raise NotImplementedError("write your pallas kernel here")


# speedup vs baseline: 4.0470x; 4.0470x over previous
import functools

import jax
import jax.numpy as jnp
from jax import lax
from jax.experimental import pallas as pl
from jax.experimental.pallas import tpu as pltpu
from jax.experimental.pallas import tpu_sc as plsc

N = 10000
F = 128
HID = 128
NTILES = 16
N_PAD = 10240
ROWS_PER_TILE = N_PAD // NTILES
CHUNK = 128
LANES = 16


def _sc_agg_body(x_hbm, h_hbm, src_hbm, dst_hbm, w_hbm, out_hbm,
                 acc, srcv, dstv, wv, rows, gsem):
    core = lax.axis_index("c")
    sid = lax.axis_index("s")
    per_tile = src_hbm.shape[0] // NTILES
    nchunks = per_tile // CHUNK

    def zrow(r, carry):
        for f2 in range(F // LANES):
            rows[r, pl.ds(f2 * LANES, LANES)] = jnp.zeros((LANES,), jnp.float32)
        return carry

    lax.fori_loop(0, CHUNK, zrow, 0)
    row0 = sid * ROWS_PER_TILE
    for kk in range(ROWS_PER_TILE // CHUNK):
        pltpu.sync_copy(rows, acc.at[pl.ds(row0 + kk * CHUNK, CHUNK)])
    plsc.subcore_barrier()

    def run_edges(z_hbm):
        base_e = sid * per_tile

        def chunk_body(k, carry):
            off = pl.multiple_of(base_e + k * CHUNK, CHUNK)
            pltpu.sync_copy(src_hbm.at[pl.ds(off, CHUNK)], srcv)
            pltpu.sync_copy(dst_hbm.at[pl.ds(off, CHUNK)], dstv)
            pltpu.sync_copy(w_hbm.at[pl.ds(off, CHUNK)], wv)
            pltpu.async_copy(z_hbm.at[srcv], rows, gsem).wait()

            def scale_row(r, c2):
                widx = jnp.full((LANES,), r, jnp.int32)
                wvec = plsc.load_gather(wv, [widx])
                for f2 in range(F // LANES):
                    sl = pl.ds(f2 * LANES, LANES)
                    rows[r, sl] = rows[r, sl] * wvec
                return c2

            lax.fori_loop(0, CHUNK, scale_row, 0)
            pltpu.sync_copy(rows, acc.at[dstv], add=True)
            return carry

        lax.fori_loop(0, nchunks, chunk_body, 0)

    @pl.when(core == 0)
    def _():
        run_edges(x_hbm)

    @pl.when(core == 1)
    def _():
        run_edges(h_hbm)

    plsc.subcore_barrier()
    out_off = core * N_PAD + row0
    pltpu.sync_copy(acc.at[pl.ds(row0, ROWS_PER_TILE)],
                    out_hbm.at[pl.ds(out_off, ROWS_PER_TILE)])


def _build_sc_agg(interpret=False):
    return pl.kernel(
        _sc_agg_body,
        out_type=jax.ShapeDtypeStruct((2 * N_PAD, F), jnp.float32),
        mesh=plsc.VectorSubcoreMesh(core_axis_name="c", subcore_axis_name="s",
                                    num_cores=2, num_subcores=NTILES),
        scratch_types=[
            pltpu.VMEM_SHARED((N_PAD, F), jnp.float32),
            pltpu.VMEM((CHUNK,), jnp.int32),
            pltpu.VMEM((CHUNK,), jnp.int32),
            pltpu.VMEM((CHUNK,), jnp.float32),
            pltpu.VMEM((CHUNK, F), jnp.float32),
            pltpu.SemaphoreType.DMA,
        ],
        compiler_params=pltpu.CompilerParams(needs_layout_passes=False),
        interpret=interpret,
    )


_SC_AGG_CACHE = []


def _sc_agg(*args):
    if not _SC_AGG_CACHE:
        _SC_AGG_CACHE.append(_build_sc_agg())
    return _SC_AGG_CACHE[0](*args)

BLK = 2000
NBLK = N // BLK


def _tc_body(x_ref, ax_ref, h_ref, ah_ref, c_ref, w_ref, b_ref, wp_ref,
             lng_ref, lnb_ref, fcw_ref, fcb_ref,
             hout_ref, cout_ref, pred_ref):
    z = jnp.concatenate(
        [x_ref[...], ax_ref[...], h_ref[...], ah_ref[...]], axis=1)
    pre = jnp.dot(z, w_ref[...], preferred_element_type=jnp.float32)
    c = c_ref[...]
    b = b_ref[...]
    wp = wp_ref[...]
    ig = jax.nn.sigmoid(pre[:, 0:HID] + wp[0:1] * c + b[0:1])
    fg = jax.nn.sigmoid(pre[:, HID:2 * HID] + wp[1:2] * c + b[1:2])
    gg = jnp.tanh(pre[:, 2 * HID:3 * HID] + b[2:3])
    c_new = fg * c + ig * gg
    og = jax.nn.sigmoid(pre[:, 3 * HID:4 * HID] + wp[2:3] * c_new + b[3:4])
    h_new = og * jnp.tanh(c_new)
    out = jax.nn.relu(h_new)
    mu = jnp.mean(out, axis=-1, keepdims=True)
    var = jnp.mean((out - mu) ** 2, axis=-1, keepdims=True)
    normed = (out - mu) * lax.rsqrt(var + 1e-5) * lng_ref[...] + lnb_ref[...]
    p = jnp.sum(normed * fcw_ref[...], axis=-1, keepdims=True) + fcb_ref[0, 0]
    pred_ref[...] = jnp.broadcast_to(jax.nn.sigmoid(p), (BLK, HID))
    hout_ref[...] = h_new
    cout_ref[...] = c_new


def _build_tc(interpret=False):
    bcast = lambda i: (0, 0)
    row_blk = lambda i: (i, 0)
    return pl.pallas_call(
        _tc_body,
        grid=(NBLK,),
        in_specs=[
            pl.BlockSpec((BLK, F), row_blk),
            pl.BlockSpec((BLK, F), row_blk),
            pl.BlockSpec((BLK, HID), row_blk),
            pl.BlockSpec((BLK, HID), row_blk),
            pl.BlockSpec((BLK, HID), row_blk),
            pl.BlockSpec((4 * F, 4 * HID), bcast),
            pl.BlockSpec((4, HID), bcast),
            pl.BlockSpec((3, HID), bcast),
            pl.BlockSpec((1, HID), bcast),
            pl.BlockSpec((1, HID), bcast),
            pl.BlockSpec((1, HID), bcast),
            pl.BlockSpec((1, 1), bcast),
        ],
        out_specs=[
            pl.BlockSpec((BLK, HID), row_blk),
            pl.BlockSpec((BLK, HID), row_blk),
            pl.BlockSpec((BLK, HID), row_blk),
        ],
        out_shape=[
            jax.ShapeDtypeStruct((N, HID), jnp.float32),
            jax.ShapeDtypeStruct((N, HID), jnp.float32),
            jax.ShapeDtypeStruct((N, HID), jnp.float32),
        ],
        interpret=interpret,
    )


_TC = _build_tc()


def kernel(X, edge_index, edge_weight, skip, H, C, Wx0, Wx1, Wh0, Wh1, b,
           w_peep, ln_g, ln_b, fc_w, fc_b):
    del skip
    x = X[0]
    h = H[0]
    c = C[0]
    e = edge_weight.shape[0]
    e_pad = -(-e // (NTILES * CHUNK)) * (NTILES * CHUNK)
    pad = e_pad - e
    src = jnp.pad(edge_index[0].astype(jnp.int32), (0, pad))
    dst = jnp.pad(edge_index[1].astype(jnp.int32), (0, pad))
    w = jnp.pad(edge_weight.astype(jnp.float32), (0, pad))

    agg = _sc_agg(x, h, src, dst, w)
    aggx = agg[:N]
    aggh = agg[N_PAD:N_PAD + N]

    w_all = jnp.concatenate([Wx0, Wx1, Wh0, Wh1], axis=1)
    w_big = jnp.transpose(w_all, (1, 0, 2)).reshape(4 * F, 4 * HID)

    h_new, c_new, pred = _TC(
        x, aggx, h, aggh, c, w_big, b, w_peep,
        ln_g.reshape(1, HID), ln_b.reshape(1, HID),
        fc_w.reshape(1, HID), fc_b.reshape(1, 1))

    return (pred[:, :1], h_new[None], c_new[None])
